# baseline (speedup 1.0000x reference)
import functools

import jax
import jax.numpy as jnp
from jax import lax
from jax.experimental import pallas as pl
from jax.experimental.pallas import tpu as pltpu
from jax.experimental.pallas import tpu_sc as plsc

NC = 2
NS = 16
NW = NC * NS
LANES = 16


def _make_kernel(B, L, V, D):
    R = B * L
    assert R % NW == 0
    RPW = R // NW
    CH = 32
    assert RPW % (2 * CH) == 0
    NCH = RPW // CH
    TROWS = V * L

    mesh = plsc.VectorSubcoreMesh(core_axis_name="c", subcore_axis_name="s")

    @functools.partial(
        pl.kernel,
        mesh=mesh,
        out_type=jax.ShapeDtypeStruct((R, D), jnp.float32),
        scratch_types=[
            pltpu.VMEM((V, D), jnp.float32),
            pltpu.VMEM((L, D), jnp.float32),
            pltpu.VMEM((TROWS, D), jnp.float32),
            pltpu.VMEM((RPW,), jnp.int32),
            pltpu.VMEM((CH, D), jnp.float32),
            pltpu.VMEM((CH, D), jnp.float32),
            pltpu.SemaphoreType.DMA,
            pltpu.SemaphoreType.DMA,
        ],
    )
    def k(x_hbm, word_hbm, pos_hbm, out_hbm,
          word_v, pos_v, table_v, idx_v, stage0, stage1, sw0, sw1):
        cid = lax.axis_index("c")
        sid = lax.axis_index("s")
        wid = sid * NC + cid
        base = wid * RPW

        pltpu.sync_copy(word_hbm, word_v)
        pltpu.sync_copy(pos_hbm, pos_v)
        idx_load = pltpu.make_async_copy(
            x_hbm.at[pl.ds(base, RPW)], idx_v, sw0)
        idx_load.start()

        def build_row(t, carry):
            v = t // L
            p = lax.rem(t, L)
            for d in range(D // LANES):
                sl = pl.ds(d * LANES, LANES)
                table_v[t, sl] = word_v[v, sl] + pos_v[p, sl]
            return carry

        lax.fori_loop(0, TROWS, build_row, 0)

        idx_load.wait()
        lane = lax.iota(jnp.int32, LANES)

        def fuse(j, carry):
            off = pl.multiple_of(j * LANES, LANES)
            sl = pl.ds(off, LANES)
            idx_v[sl] = idx_v[sl] * L + lax.rem(off + lane, L)
            return carry

        lax.fori_loop(0, RPW // LANES, fuse, 0)

        def expand(ci, buf):
            def grp(j, carry):
                off = pl.multiple_of(ci * CH + j * LANES, LANES)
                vv = idx_v[pl.ds(off, LANES)]
                for i in range(LANES):
                    v = vv[i]
                    r = j * LANES + i
                    for d in range(D // LANES):
                        sl = pl.ds(d * LANES, LANES)
                        buf[r, sl] = table_v[v, sl]
                return carry

            lax.fori_loop(0, CH // LANES, grp, 0)

        def w_copy(ci, buf, sem):
            off = pl.multiple_of(ci * CH, CH)
            return pltpu.make_async_copy(
                buf, out_hbm.at[pl.ds(base + off, CH)], sem)

        expand(0, stage0)
        w_copy(0, stage0, sw0).start()
        expand(1, stage1)
        w_copy(1, stage1, sw1).start()

        def pipe(g, carry):
            c0 = 2 * g + 2
            c1 = 2 * g + 3
            w_copy(c0 - 2, stage0, sw0).wait()
            expand(c0, stage0)
            w_copy(c0, stage0, sw0).start()
            w_copy(c1 - 2, stage1, sw1).wait()
            expand(c1, stage1)
            w_copy(c1, stage1, sw1).start()
            return carry

        lax.fori_loop(0, (NCH - 2) // 2, pipe, 0)
        w_copy(NCH - 2, stage0, sw0).wait()
        w_copy(NCH - 1, stage1, sw1).wait()

    return k


def kernel(X, word_embedding, pos_embedding):
    B, L = X.shape
    V, D = word_embedding.shape
    k = _make_kernel(B, L, V, D)
    x_flat = X.reshape(-1).astype(jnp.int32)
    out = k(x_flat, word_embedding, pos_embedding)
    return out.reshape(B, L, D)


# speedup vs baseline: 1.6682x; 1.6682x over previous
import functools

import jax
import jax.numpy as jnp
from jax import lax
from jax.experimental import pallas as pl
from jax.experimental.pallas import tpu as pltpu
from jax.experimental.pallas import tpu_sc as plsc

NC = 2
NS = 16
NW = NC * NS
LANES = 16


def _make_kernel(B, L, V, D):
    R = B * L
    assert R % NW == 0
    RPW = R // NW
    BPW = B // NW
    CB = 2
    CH = CB * L
    assert CH % 8 == 0 and BPW % (2 * CB) == 0
    NCH = BPW // CB
    TROWS = V * L

    mesh = plsc.VectorSubcoreMesh(core_axis_name="c", subcore_axis_name="s")

    @functools.partial(
        pl.kernel,
        mesh=mesh,
        out_type=jax.ShapeDtypeStruct((B, L, D), jnp.float32),
        scratch_types=[
            pltpu.VMEM((V, D), jnp.float32),
            pltpu.VMEM((L, D), jnp.float32),
            pltpu.VMEM((TROWS, D), jnp.float32),
            pltpu.VMEM((RPW,), jnp.int32),
            pltpu.VMEM((CB, L, D), jnp.float32),
            pltpu.VMEM((CB, L, D), jnp.float32),
            pltpu.SemaphoreType.DMA,
            pltpu.SemaphoreType.DMA,
        ],
    )
    def k(x_hbm, word_hbm, pos_hbm, out_hbm,
          word_v, pos_v, table_v, idx_v, stage0, stage1, sw0, sw1):
        cid = lax.axis_index("c")
        sid = lax.axis_index("s")
        wid = sid * NC + cid
        base = wid * RPW
        bbase = wid * BPW

        pltpu.sync_copy(word_hbm, word_v)
        pltpu.sync_copy(pos_hbm, pos_v)
        idx_load = pltpu.make_async_copy(
            x_hbm.at[pl.ds(base, RPW)], idx_v, sw0)
        idx_load.start()

        ND = D // LANES
        sls = [pl.ds(d * LANES, LANES) for d in range(ND)]

        def build_row(t, carry):
            v = t // L
            p = lax.rem(t, L)
            ws = [word_v[v, sl] for sl in sls]
            ps = [pos_v[p, sl] for sl in sls]
            for d in range(ND):
                table_v[t, sls[d]] = ws[d] + ps[d]
            return carry

        lax.fori_loop(0, TROWS, build_row, 0)

        idx_load.wait()
        lane = lax.iota(jnp.int32, LANES)

        def fuse(j, carry):
            off = pl.multiple_of(j * LANES, LANES)
            sl = pl.ds(off, LANES)
            idx_v[sl] = idx_v[sl] * L + lax.rem(off + lane, L)
            return carry

        lax.fori_loop(0, RPW // LANES, fuse, 0)

        def expand(ci, buf):
            off = pl.multiple_of(ci * CH, 8)
            vv = [idx_v[pl.ds(off, LANES)],
                  idx_v[pl.ds(off + CH - LANES, LANES)]]

            def vget(i):
                return vv[0][i] if i < LANES else vv[1][i - (CH - LANES)]

            prev = [table_v[vget(0), sl] for sl in sls]
            for i in range(1, CH + 1):
                cur = ([table_v[vget(i), sl] for sl in sls]
                       if i < CH else None)
                r = i - 1
                for d in range(ND):
                    buf[r // L, r % L, sls[d]] = prev[d]
                prev = cur

        def w_copy(ci, buf, sem):
            off = ci * CB
            return pltpu.make_async_copy(
                buf, out_hbm.at[pl.ds(bbase + off, CB)], sem)

        expand(0, stage0)
        w_copy(0, stage0, sw0).start()
        expand(1, stage1)
        w_copy(1, stage1, sw1).start()

        def pipe(g, carry):
            c0 = 2 * g + 2
            c1 = 2 * g + 3
            w_copy(c0 - 2, stage0, sw0).wait()
            expand(c0, stage0)
            w_copy(c0, stage0, sw0).start()
            w_copy(c1 - 2, stage1, sw1).wait()
            expand(c1, stage1)
            w_copy(c1, stage1, sw1).start()
            return carry

        lax.fori_loop(0, (NCH - 2) // 2, pipe, 0)
        w_copy(NCH - 2, stage0, sw0).wait()
        w_copy(NCH - 1, stage1, sw1).wait()

    return k


def kernel(X, word_embedding, pos_embedding):
    B, L = X.shape
    V, D = word_embedding.shape
    k = _make_kernel(B, L, V, D)
    x_flat = X.reshape(-1).astype(jnp.int32)
    return k(x_flat, word_embedding, pos_embedding)
